# Initial kernel scaffold; baseline (speedup 1.0000x reference)
import functools

import jax
import jax.numpy as jnp
from jax import lax
from jax.experimental import pallas as pl
from jax.experimental.pallas import tpu as pltpu
from jax.experimental.pallas import tpu_sc as plsc

NC = 2
NS = 16
NW = NC * NS
K = 80
RZ = 125


def _mm_body(z_ref, w_ref, bias_ref, o_ref):
    o_ref[...] = lax.dot_general(
        z_ref[...], w_ref[...], (((1,), (1,)), ((), ())),
        preferred_element_type=jnp.float32) + bias_ref[...]


def _merge_mm_body(a0_ref, a1_ref, gb_ref, al_ref, w_ref, bias_ref, o_ref):
    x = a0_ref[...] + a1_ref[...] + gb_ref[...]
    z = jnp.where(x > 0, x, al_ref[...] * x)
    o_ref[...] = lax.dot_general(
        z, w_ref[...], (((1,), (1,)), ((), ())),
        preferred_element_type=jnp.float32) + bias_ref[...]


def _merge_body(a0_ref, a1_ref, gb_ref, al_ref, o_ref):
    x = a0_ref[...] + a1_ref[...] + gb_ref[...]
    o_ref[...] = jnp.where(x > 0, x, al_ref[...] * x)


def _row_spec(bm, d):
    return pl.BlockSpec((bm, d), lambda i: (i, 0))


def _full_spec(r, c):
    return pl.BlockSpec((r, c), lambda i: (0, 0))


def _mm(z, w, bias, bm=1000):
    m, d = z.shape
    return pl.pallas_call(
        _mm_body,
        grid=(m // bm,),
        in_specs=[_row_spec(bm, d), _full_spec(d, d), _full_spec(1, d)],
        out_specs=_row_spec(bm, d),
        out_shape=jax.ShapeDtypeStruct((m, d), jnp.float32),
    )(z, w, bias)


def _merge_mm(a0, a1, gb, al, w, bias, bm=1000):
    m, d = a0.shape
    return pl.pallas_call(
        _merge_mm_body,
        grid=(m // bm,),
        in_specs=[_row_spec(bm, d), _row_spec(bm, d), _full_spec(1, d),
                  _full_spec(1, d), _full_spec(d, d), _full_spec(1, d)],
        out_specs=_row_spec(bm, d),
        out_shape=jax.ShapeDtypeStruct((m, d), jnp.float32),
    )(a0, a1, gb, al, w, bias)


def _merge(a0, a1, gb, al, bm=1000):
    m, d = a0.shape
    return pl.pallas_call(
        _merge_body,
        grid=(m // bm,),
        in_specs=[_row_spec(bm, d), _row_spec(bm, d), _full_spec(1, d),
                  _full_spec(1, d)],
        out_specs=_row_spec(bm, d),
        out_shape=jax.ShapeDtypeStruct((m, d), jnp.float32),
    )(a0, a1, gb, al)


@functools.cache
def _make_sc_scatter(n, d, e):
    c = e // (NW * K)
    assert c * NW * K == e
    rps = n // NS
    assert rps % RZ == 0
    mesh = plsc.VectorSubcoreMesh(
        core_axis_name="c", subcore_axis_name="s",
        num_cores=NC, num_subcores=NS)

    @functools.partial(
        pl.kernel,
        out_type=(jax.ShapeDtypeStruct((n, d), jnp.float32),
                  jax.ShapeDtypeStruct((n, d), jnp.float32)),
        mesh=mesh,
        scratch_types=[
            pltpu.VMEM((c, K), jnp.int32),
            pltpu.VMEM((c, K), jnp.int32),
            pltpu.VMEM((K, d), jnp.float32),
            pltpu.VMEM((RZ, d), jnp.float32),
            pltpu.VMEM_SHARED((n, d), jnp.float32),
            pltpu.SemaphoreType.DMA,
        ],
    )
    def sc_scatter(fts_hbm, src_hbm, dst_hbm, out0, out1,
                   src_v, dst_v, rows_v, zbuf, acc, sem):
        cid = lax.axis_index("c")
        sid = lax.axis_index("s")
        wid = sid * NC + cid

        def zrow(r, carry):
            for cc in range(d // 16):
                zbuf[r, pl.ds(cc * 16, 16)] = jnp.zeros((16,), jnp.float32)
            return carry
        lax.fori_loop(0, RZ, zrow, 0)
        for t in range(rps // RZ):
            pltpu.sync_copy(zbuf, acc.at[pl.ds(sid * rps + t * RZ, RZ)])

        pltpu.sync_copy(src_hbm.at[wid], src_v)
        pltpu.sync_copy(dst_hbm.at[wid], dst_v)
        plsc.subcore_barrier()

        def body(j, carry):
            pltpu.async_copy(fts_hbm.at[src_v.at[j]], rows_v, sem).wait()
            pltpu.sync_copy(rows_v, acc.at[dst_v.at[j]], add=True)
            return carry
        lax.fori_loop(0, c, body, 0)
        plsc.subcore_barrier()

        for t in range(rps // RZ):
            base = sid * rps + t * RZ
            pltpu.sync_copy(acc.at[pl.ds(base, RZ)], zbuf)

            @pl.when(cid == 0)
            def _():
                pltpu.sync_copy(zbuf, out0.at[pl.ds(base, RZ)])

            @pl.when(cid == 1)
            def _():
                pltpu.sync_copy(zbuf, out1.at[pl.ds(base, RZ)])

    return sc_scatter


def kernel(h, edge_index, W, b, gcn_bias, alpha):
    n, d = h.shape
    e = edge_index.shape[1]
    l = W.shape[0]
    c = e // (NW * K)

    src = edge_index[0].reshape(NW, c, K)
    dst = edge_index[1].reshape(NW, c, K)
    b2 = b.reshape(l, 1, d)
    gb2 = gcn_bias.reshape(l, 1, d)
    al2 = jnp.broadcast_to(alpha.reshape(l, 1, 1), (l, 1, d))

    sc_scatter = _make_sc_scatter(n, d, e)

    a0 = a1 = None
    for i in range(l):
        if i == 0:
            fts = _mm(h, W[0], b2[0])
        else:
            fts = _merge_mm(a0, a1, gb2[i - 1], al2[i - 1], W[i], b2[i])
        a0, a1 = sc_scatter(fts, src, dst)
    return _merge(a0, a1, gb2[l - 1], al2[l - 1])


# speedup vs baseline: 4.6233x; 4.6233x over previous
import functools

import jax
import jax.numpy as jnp
from jax import lax
from jax.experimental import pallas as pl
from jax.experimental.pallas import tpu as pltpu
from jax.experimental.pallas import tpu_sc as plsc

NC = 2
NS = 16
K = 80
RZ = 128


def _mm_body(z_ref, w_ref, bias_ref, o_ref):
    o_ref[...] = (lax.dot_general(
        z_ref[...], w_ref[...], (((1,), (1,)), ((), ())),
        preferred_element_type=jnp.float32) + bias_ref[0])[None]


def _merge_mm_body(a_ref, gb_ref, al_ref, w_ref, bias_ref, o_ref):
    x = jnp.concatenate([a_ref[0], a_ref[1]], axis=1) + gb_ref[...]
    z = jnp.where(x > 0, x, al_ref[...] * x)
    o_ref[...] = (lax.dot_general(
        z, w_ref[...], (((1,), (1,)), ((), ())),
        preferred_element_type=jnp.float32) + bias_ref[0])[None]


def _merge_body(a_ref, gb_ref, al_ref, o_ref):
    x = jnp.concatenate([a_ref[0], a_ref[1]], axis=1) + gb_ref[...]
    o_ref[...] = jnp.where(x > 0, x, al_ref[...] * x)


def _mm(z, w, bias, bm=1000):
    m, d = z.shape
    dh = d // 2
    return pl.pallas_call(
        _mm_body,
        grid=(m // bm, 2),
        in_specs=[pl.BlockSpec((bm, d), lambda i, j: (i, 0)),
                  pl.BlockSpec((dh, d), lambda i, j: (j, 0)),
                  pl.BlockSpec((1, 1, dh), lambda i, j: (j, 0, 0))],
        out_specs=pl.BlockSpec((1, bm, dh), lambda i, j: (j, i, 0)),
        out_shape=jax.ShapeDtypeStruct((2, m, dh), jnp.float32),
    )(z, w, bias)


def _merge_mm(a, gb, al, w, bias, m, bm=1000):
    d = 2 * a.shape[2]
    dh = d // 2
    return pl.pallas_call(
        _merge_mm_body,
        grid=(m // bm, 2),
        in_specs=[pl.BlockSpec((2, bm, dh), lambda i, j: (0, i, 0)),
                  pl.BlockSpec((1, d), lambda i, j: (0, 0)),
                  pl.BlockSpec((1, d), lambda i, j: (0, 0)),
                  pl.BlockSpec((dh, d), lambda i, j: (j, 0)),
                  pl.BlockSpec((1, 1, dh), lambda i, j: (j, 0, 0))],
        out_specs=pl.BlockSpec((1, bm, dh), lambda i, j: (j, i, 0)),
        out_shape=jax.ShapeDtypeStruct((2, m, dh), jnp.float32),
    )(a, gb, al, w, bias)


def _merge(a, gb, al, m, bm=1000):
    d = 2 * a.shape[2]
    dh = d // 2
    return pl.pallas_call(
        _merge_body,
        grid=(m // bm,),
        in_specs=[pl.BlockSpec((2, bm, dh), lambda i: (0, i, 0)),
                  pl.BlockSpec((1, d), lambda i: (0, 0)),
                  pl.BlockSpec((1, d), lambda i: (0, 0))],
        out_specs=pl.BlockSpec((bm, d), lambda i: (i, 0)),
        out_shape=jax.ShapeDtypeStruct((m, d), jnp.float32),
    )(a, gb, al)


@functools.cache
def _make_sc_scatter(n, dh, e):
    c = e // (NS * K)
    assert c * NS * K == e
    rps = -(-n // (NS * RZ)) * RZ
    n_pad = rps * NS
    mesh = plsc.VectorSubcoreMesh(
        core_axis_name="c", subcore_axis_name="s",
        num_cores=NC, num_subcores=NS)

    @functools.partial(
        pl.kernel,
        out_type=jax.ShapeDtypeStruct((NC, n_pad, dh), jnp.float32),
        mesh=mesh,
        scratch_types=[
            pltpu.VMEM((c, K), jnp.int32),
            pltpu.VMEM((c, K), jnp.int32),
            pltpu.VMEM((K, dh), jnp.float32),
            pltpu.VMEM((RZ, dh), jnp.float32),
            pltpu.VMEM_SHARED((n_pad, dh), jnp.float32),
            pltpu.SemaphoreType.DMA,
        ],
        compiler_params=pltpu.CompilerParams(use_tc_tiling_on_sc=False),
    )
    def sc_scatter(fts_hbm, src_hbm, dst_hbm, out,
                   src_v, dst_v, rows_v, zbuf, acc, sem):
        cid = lax.axis_index("c")
        sid = lax.axis_index("s")

        def zrow(r, carry):
            for cc in range(dh // 16):
                zbuf[r, pl.ds(cc * 16, 16)] = jnp.zeros((16,), jnp.float32)
            return carry
        lax.fori_loop(0, RZ, zrow, 0)
        for t in range(rps // RZ):
            pltpu.sync_copy(zbuf, acc.at[pl.ds(sid * rps + t * RZ, RZ)])

        pltpu.sync_copy(src_hbm.at[cid, sid], src_v)
        pltpu.sync_copy(dst_hbm.at[sid], dst_v)
        plsc.subcore_barrier()

        def body(j, carry):
            pltpu.async_copy(fts_hbm.at[src_v.at[j]], rows_v, sem).wait()
            pltpu.sync_copy(rows_v, acc.at[dst_v.at[j]], add=True)
            return carry
        lax.fori_loop(0, c, body, 0)
        plsc.subcore_barrier()

        for t in range(rps // RZ):
            base = sid * rps + t * RZ
            pltpu.sync_copy(acc.at[pl.ds(base, RZ)], zbuf)
            pltpu.sync_copy(zbuf, out.at[cid, pl.ds(base, RZ)])

    return sc_scatter


def kernel(h, edge_index, W, b, gcn_bias, alpha):
    n, d = h.shape
    e = edge_index.shape[1]
    l = W.shape[0]
    dh = d // 2
    c = e // (NS * K)

    src = edge_index[0]
    dst = edge_index[1]
    srcs = jnp.stack([src, src + n]).reshape(2, NS, c, K)
    dsts = dst.reshape(NS, c, K)
    b2 = b.reshape(l, 2, 1, dh)
    gb2 = gcn_bias.reshape(l, 1, d)
    al2 = jnp.broadcast_to(alpha.reshape(l, 1, 1), (l, 1, d))

    sc_scatter = _make_sc_scatter(n, dh, e)

    a = None
    for i in range(l):
        if i == 0:
            fts = _mm(h, W[0], b2[0])
        else:
            fts = _merge_mm(a, gb2[i - 1], al2[i - 1], W[i], b2[i], n)
        a = sc_scatter(fts.reshape(2 * n, dh), srcs, dsts)
    return _merge(a, gb2[l - 1], al2[l - 1], n)


# speedup vs baseline: 7.3581x; 1.5915x over previous
import functools

import jax
import jax.numpy as jnp
from jax import lax
from jax.experimental import pallas as pl
from jax.experimental.pallas import tpu as pltpu
from jax.experimental.pallas import tpu_sc as plsc

NC = 2
NS = 16
K = 80
RZ = 128


def _mm_body(z_ref, w_ref, bias_ref, o_ref):
    o_ref[...] = (lax.dot_general(
        z_ref[...], w_ref[...], (((1,), (1,)), ((), ())),
        preferred_element_type=jnp.float32) + bias_ref[0])[None]


def _merge_mm_body(a_ref, gb_ref, al_ref, w_ref, bias_ref, o_ref):
    x = jnp.concatenate([a_ref[0], a_ref[1]], axis=1) + gb_ref[...]
    z = jnp.where(x > 0, x, al_ref[...] * x)
    o_ref[...] = (lax.dot_general(
        z, w_ref[...], (((1,), (1,)), ((), ())),
        preferred_element_type=jnp.float32) + bias_ref[0])[None]


def _merge_body(a_ref, gb_ref, al_ref, o_ref):
    x = jnp.concatenate([a_ref[0], a_ref[1]], axis=1) + gb_ref[...]
    o_ref[...] = jnp.where(x > 0, x, al_ref[...] * x)


def _mm(z, w, bias, bm=1000):
    m, d = z.shape
    dh = d // 2
    return pl.pallas_call(
        _mm_body,
        grid=(m // bm, 2),
        in_specs=[pl.BlockSpec((bm, d), lambda i, j: (i, 0)),
                  pl.BlockSpec((dh, d), lambda i, j: (j, 0)),
                  pl.BlockSpec((1, 1, dh), lambda i, j: (j, 0, 0))],
        out_specs=pl.BlockSpec((1, bm, dh), lambda i, j: (j, i, 0)),
        out_shape=jax.ShapeDtypeStruct((2, m, dh), jnp.float32),
    )(z, w, bias)


def _merge_mm(a, gb, al, w, bias, m, bm=1000):
    d = 2 * a.shape[2]
    dh = d // 2
    return pl.pallas_call(
        _merge_mm_body,
        grid=(m // bm, 2),
        in_specs=[pl.BlockSpec((2, bm, dh), lambda i, j: (0, i, 0)),
                  pl.BlockSpec((1, d), lambda i, j: (0, 0)),
                  pl.BlockSpec((1, d), lambda i, j: (0, 0)),
                  pl.BlockSpec((dh, d), lambda i, j: (j, 0)),
                  pl.BlockSpec((1, 1, dh), lambda i, j: (j, 0, 0))],
        out_specs=pl.BlockSpec((1, bm, dh), lambda i, j: (j, i, 0)),
        out_shape=jax.ShapeDtypeStruct((2, m, dh), jnp.float32),
    )(a, gb, al, w, bias)


def _merge(a, gb, al, m, bm=1000):
    d = 2 * a.shape[2]
    dh = d // 2
    return pl.pallas_call(
        _merge_body,
        grid=(m // bm,),
        in_specs=[pl.BlockSpec((2, bm, dh), lambda i: (0, i, 0)),
                  pl.BlockSpec((1, d), lambda i: (0, 0)),
                  pl.BlockSpec((1, d), lambda i: (0, 0))],
        out_specs=pl.BlockSpec((bm, d), lambda i: (i, 0)),
        out_shape=jax.ShapeDtypeStruct((m, d), jnp.float32),
    )(a, gb, al)


@functools.cache
def _make_sc_scatter(n, dh, e):
    c = e // (NS * K)
    assert c * NS * K == e
    rps = -(-n // (NS * RZ)) * RZ
    n_pad = rps * NS
    mesh = plsc.VectorSubcoreMesh(
        core_axis_name="c", subcore_axis_name="s",
        num_cores=NC, num_subcores=NS)

    @functools.partial(
        pl.kernel,
        out_type=jax.ShapeDtypeStruct((NC, n_pad, dh), jnp.float32),
        mesh=mesh,
        scratch_types=[
            pltpu.VMEM((c, K), jnp.int32),
            pltpu.VMEM((c, K), jnp.int32),
            pltpu.VMEM((K, dh), jnp.float32),
            pltpu.VMEM((K, dh), jnp.float32),
            pltpu.VMEM((RZ, dh), jnp.float32),
            pltpu.VMEM_SHARED((n_pad, dh), jnp.float32),
            pltpu.SemaphoreType.DMA,
            pltpu.SemaphoreType.DMA,
        ],
        compiler_params=pltpu.CompilerParams(use_tc_tiling_on_sc=False),
    )
    def sc_scatter(fts_hbm, src_hbm, dst_hbm, out,
                   src_v, dst_v, rows_a, rows_b, zbuf, acc, sem_a, sem_b):
        cid = lax.axis_index("c")
        sid = lax.axis_index("s")

        def zrow(r, carry):
            for cc in range(dh // 16):
                zbuf[r, pl.ds(cc * 16, 16)] = jnp.zeros((16,), jnp.float32)
            return carry
        lax.fori_loop(0, RZ, zrow, 0)
        for t in range(rps // RZ):
            pltpu.sync_copy(zbuf, acc.at[pl.ds(sid * rps + t * RZ, RZ)])

        pltpu.sync_copy(src_hbm.at[cid, sid], src_v)
        pltpu.sync_copy(dst_hbm.at[sid], dst_v)
        plsc.subcore_barrier()

        pltpu.async_copy(fts_hbm.at[src_v.at[0]], rows_a, sem_a)
        def body(g, carry):
            base = 2 * g
            pltpu.async_copy(fts_hbm.at[src_v.at[base + 1]], rows_b, sem_b)
            pltpu.make_async_copy(fts_hbm.at[src_v.at[base]], rows_a, sem_a).wait()
            pltpu.sync_copy(rows_a, acc.at[dst_v.at[base]], add=True)
            nxt = jnp.minimum(base + 2, c - 1)
            pltpu.async_copy(fts_hbm.at[src_v.at[nxt]], rows_a, sem_a)
            pltpu.make_async_copy(fts_hbm.at[src_v.at[base + 1]], rows_b, sem_b).wait()
            pltpu.sync_copy(rows_b, acc.at[dst_v.at[base + 1]], add=True)
            return carry
        lax.fori_loop(0, c // 2, body, 0)
        pltpu.make_async_copy(fts_hbm.at[src_v.at[c - 1]], rows_a, sem_a).wait()
        plsc.subcore_barrier()

        for t in range(rps // RZ):
            base = sid * rps + t * RZ
            pltpu.sync_copy(acc.at[pl.ds(base, RZ)], zbuf)
            pltpu.sync_copy(zbuf, out.at[cid, pl.ds(base, RZ)])

    return sc_scatter


def kernel(h, edge_index, W, b, gcn_bias, alpha):
    n, d = h.shape
    e = edge_index.shape[1]
    l = W.shape[0]
    dh = d // 2
    c = e // (NS * K)

    src = edge_index[0]
    dst = edge_index[1]
    srcs = jnp.stack([src, src + n]).reshape(2, NS, c, K)
    dsts = dst.reshape(NS, c, K)
    b2 = b.reshape(l, 2, 1, dh)
    gb2 = gcn_bias.reshape(l, 1, d)
    al2 = jnp.broadcast_to(alpha.reshape(l, 1, 1), (l, 1, d))

    sc_scatter = _make_sc_scatter(n, dh, e)

    a = None
    for i in range(l):
        if i == 0:
            fts = _mm(h, W[0], b2[0])
        else:
            fts = _merge_mm(a, gb2[i - 1], al2[i - 1], W[i], b2[i], n)
        a = sc_scatter(fts.reshape(2 * n, dh), srcs, dsts)
    return _merge(a, gb2[l - 1], al2[l - 1], n)


# speedup vs baseline: 9.0042x; 1.2237x over previous
import functools

import jax
import jax.numpy as jnp
from jax import lax
from jax.experimental import pallas as pl
from jax.experimental.pallas import tpu as pltpu
from jax.experimental.pallas import tpu_sc as plsc

NC = 2
NS = 16
K = 80
G = 5
RZ = 128


def _mm_body(z_ref, w_ref, bias_ref, o_ref):
    o_ref[...] = (lax.dot_general(
        z_ref[...], w_ref[...], (((1,), (1,)), ((), ())),
        preferred_element_type=jnp.float32) + bias_ref[0])[None]


def _merge_mm_body(a_ref, gb_ref, al_ref, w_ref, bias_ref, o_ref):
    x = jnp.concatenate([a_ref[0], a_ref[1]], axis=1) + gb_ref[...]
    z = jnp.where(x > 0, x, al_ref[...] * x)
    o_ref[...] = (lax.dot_general(
        z, w_ref[...], (((1,), (1,)), ((), ())),
        preferred_element_type=jnp.float32) + bias_ref[0])[None]


def _mm(z, w, bias, bm=2000):
    m, d = z.shape
    dh = d // 2
    return pl.pallas_call(
        _mm_body,
        grid=(m // bm, 2),
        in_specs=[pl.BlockSpec((bm, d), lambda i, j: (i, 0)),
                  pl.BlockSpec((dh, d), lambda i, j: (j, 0)),
                  pl.BlockSpec((1, 1, dh), lambda i, j: (j, 0, 0))],
        out_specs=pl.BlockSpec((1, bm, dh), lambda i, j: (j, i, 0)),
        out_shape=jax.ShapeDtypeStruct((2, m, dh), jnp.float32),
    )(z, w, bias)


def _merge_mm(a, gb, al, w, bias, m, bm=2000):
    d = 2 * a.shape[2]
    dh = d // 2
    return pl.pallas_call(
        _merge_mm_body,
        grid=(m // bm, 2),
        in_specs=[pl.BlockSpec((2, bm, dh), lambda i, j: (0, i, 0)),
                  pl.BlockSpec((1, d), lambda i, j: (0, 0)),
                  pl.BlockSpec((1, d), lambda i, j: (0, 0)),
                  pl.BlockSpec((dh, d), lambda i, j: (j, 0)),
                  pl.BlockSpec((1, 1, dh), lambda i, j: (j, 0, 0))],
        out_specs=pl.BlockSpec((1, bm, dh), lambda i, j: (j, i, 0)),
        out_shape=jax.ShapeDtypeStruct((2, m, dh), jnp.float32),
    )(a, gb, al, w, bias)


@functools.cache
def _make_sc_scatter(n, dh, e, fuse_prelu):
    c = e // (NS * K)
    assert c * NS * K == e and c % G == 0
    rps = -(-n // (NS * RZ)) * RZ
    n_pad = rps * NS
    mesh = plsc.VectorSubcoreMesh(
        core_axis_name="c", subcore_axis_name="s",
        num_cores=NC, num_subcores=NS)

    scratch = [
        pltpu.VMEM((c, K), jnp.int32),
        pltpu.VMEM((c, K), jnp.int32),
        [pltpu.VMEM((K, dh), jnp.float32)] * G,
        pltpu.VMEM((RZ, dh), jnp.float32),
        pltpu.VMEM((dh,), jnp.float32),
        pltpu.VMEM((dh,), jnp.float32),
        pltpu.VMEM_SHARED((n_pad, dh), jnp.float32),
        [pltpu.SemaphoreType.DMA] * G,
        [pltpu.SemaphoreType.DMA] * G,
    ]

    @functools.partial(
        pl.kernel,
        out_type=jax.ShapeDtypeStruct((NC, n_pad, dh), jnp.float32),
        mesh=mesh,
        scratch_types=scratch,
        compiler_params=pltpu.CompilerParams(use_tc_tiling_on_sc=False),
    )
    def sc_scatter(fts_hbm, src_hbm, dst_hbm, gb_hbm, al_hbm, out,
                   src_v, dst_v, rows, zbuf, gbv, alv, acc, gsems, ssems):
        cid = lax.axis_index("c")
        sid = lax.axis_index("s")

        def zrow(r, carry):
            for cc in range(dh // 16):
                zbuf[r, pl.ds(cc * 16, 16)] = jnp.zeros((16,), jnp.float32)
            return carry
        lax.fori_loop(0, RZ, zrow, 0)
        for t in range(rps // RZ):
            pltpu.sync_copy(zbuf, acc.at[pl.ds(sid * rps + t * RZ, RZ)])

        pltpu.sync_copy(src_hbm.at[cid, sid], src_v)
        pltpu.sync_copy(dst_hbm.at[sid], dst_v)
        if fuse_prelu:
            pltpu.sync_copy(gb_hbm.at[cid], gbv)
            pltpu.sync_copy(al_hbm.at[cid], alv)
        plsc.subcore_barrier()

        for b in range(G):
            pltpu.async_copy(fts_hbm.at[src_v.at[b]], rows[b], gsems[b])

        def body(g, carry):
            base = G * g
            for b in range(G):
                pltpu.make_async_copy(
                    fts_hbm.at[src_v.at[base + b]], rows[b], gsems[b]).wait()
                pltpu.async_copy(
                    rows[b], acc.at[dst_v.at[base + b]], ssems[b], add=True)
            nxt = jnp.minimum(base + G, c - G)
            for b in range(G):
                pltpu.make_async_copy(
                    rows[b], acc.at[dst_v.at[base + b]], ssems[b]).wait()
                pltpu.async_copy(fts_hbm.at[src_v.at[nxt + b]], rows[b],
                                 gsems[b])
            return carry
        lax.fori_loop(0, c // G, body, 0)
        for b in range(G):
            pltpu.make_async_copy(
                fts_hbm.at[src_v.at[b]], rows[b], gsems[b]).wait()
        plsc.subcore_barrier()

        for t in range(rps // RZ):
            base = sid * rps + t * RZ
            pltpu.sync_copy(acc.at[pl.ds(base, RZ)], zbuf)
            if fuse_prelu:
                def prow(r, carry):
                    for cc in range(dh // 16):
                        sl = pl.ds(cc * 16, 16)
                        x = zbuf[r, sl] + gbv[sl]
                        zbuf[r, sl] = jnp.where(x > 0, x, alv[sl] * x)
                    return carry
                lax.fori_loop(0, RZ, prow, 0)
            pltpu.sync_copy(zbuf, out.at[cid, pl.ds(base, RZ)])

    return sc_scatter


def kernel(h, edge_index, W, b, gcn_bias, alpha):
    n, d = h.shape
    e = edge_index.shape[1]
    l = W.shape[0]
    dh = d // 2
    c = e // (NS * K)

    src = edge_index[0]
    dst = edge_index[1]
    srcs = jnp.stack([src, src + n]).reshape(2, NS, c, K)
    dsts = dst.reshape(NS, c, K)
    b2 = b.reshape(l, 2, 1, dh)
    gb2 = gcn_bias.reshape(l, 1, d)
    al2 = jnp.broadcast_to(alpha.reshape(l, 1, 1), (l, 1, d))
    gb_h = gcn_bias.reshape(l, 2, dh)
    al_h = jnp.broadcast_to(alpha.reshape(l, 1, 1), (l, 2, dh))

    sc_mid = _make_sc_scatter(n, dh, e, False)
    sc_last = _make_sc_scatter(n, dh, e, True)

    a = None
    for i in range(l):
        if i == 0:
            fts = _mm(h, W[0], b2[0])
        else:
            fts = _merge_mm(a, gb2[i - 1], al2[i - 1], W[i], b2[i], n)
        sc = sc_last if i == l - 1 else sc_mid
        a = sc(fts.reshape(2 * n, dh), srcs, dsts, gb_h[i], al_h[i])
    return jnp.concatenate([a[0, :n], a[1, :n]], axis=1)
